# baseline (speedup 1.0000x reference)
import functools

import jax
import jax.numpy as jnp
from jax import lax
from jax.experimental import pallas as pl
from jax.experimental.pallas import tpu as pltpu
from jax.experimental.pallas import tpu_sc as plsc

B = 2
N_PIX = 147456
D = 128
N_MATCH = 5000
N_NONMATCH = 20000
MARGIN = 0.5
NONMATCH_W = 1.0

NC = 2
NS = 16
NW = NC * NS
L = 16

M_TOT = B * N_MATCH
NM_TOT = B * N_NONMATCH
M_PER_W = -(-M_TOT // NW)
NM_PER_W = NM_TOT // NW

F_M = 320
F_NM = 1264
ROW_MAX = B * N_PIX - 1

T = 128


def _tiles(total):
    out, s = [], 0
    while s < total:
        out.append((s, min(T, total - s)))
        s += T
    return out


M_TILES = _tiles(M_PER_W)
NM_TILES = _tiles(NM_PER_W)

_mesh = plsc.VectorSubcoreMesh(core_axis_name="c", subcore_axis_name="s")


@functools.partial(
    pl.kernel,
    out_type=jax.ShapeDtypeStruct((NW, 2 * L), jnp.float32),
    mesh=_mesh,
    scratch_types=[
        pltpu.VMEM((F_M + 2 * L,), jnp.int32),
        pltpu.VMEM((F_M + 2 * L,), jnp.int32),
        pltpu.VMEM((F_NM + L,), jnp.int32),
        pltpu.VMEM((F_NM + L,), jnp.int32),
        pltpu.VMEM((F_M,), jnp.int32),
        pltpu.VMEM((F_M,), jnp.int32),
        pltpu.VMEM((F_NM,), jnp.int32),
        pltpu.VMEM((F_NM,), jnp.int32),
        pltpu.VMEM((T, D), jnp.float32),
        pltpu.VMEM((T, D), jnp.float32),
        pltpu.VMEM((T, D), jnp.float32),
        pltpu.VMEM((T, D), jnp.float32),
        pltpu.VMEM((2 * L,), jnp.float32),
        pltpu.SemaphoreType.DMA,
        pltpu.SemaphoreType.DMA,
        pltpu.SemaphoreType.DMA,
    ],
)
def _sc_loss(tableA, tableB, mA, mB, nmA, nmB, out,
             winMA, winMB, winNA, winNB,
             iAm_v, iBm_v, iAnm_v, iBnm_v,
             bufA0, bufB0, bufA1, bufB1, stage, sem0, sem1, sem_i):
    wid = lax.axis_index("s") * NC + lax.axis_index("c")

    base_m = wid * M_PER_W
    astart_m = jnp.minimum((base_m // 8) * 8, M_TOT - F_M)
    off_m = base_m - astart_m
    base_nm = wid * NM_PER_W
    astart_nm = jnp.minimum((base_nm // 8) * 8, NM_TOT - F_NM)
    off_nm = base_nm - astart_nm

    cps = [pltpu.async_copy(mA.at[pl.ds(astart_m, F_M)],
                            winMA.at[pl.ds(0, F_M)], sem_i),
           pltpu.async_copy(mB.at[pl.ds(astart_m, F_M)],
                            winMB.at[pl.ds(0, F_M)], sem_i),
           pltpu.async_copy(nmA.at[pl.ds(astart_nm, F_NM)],
                            winNA.at[pl.ds(0, F_NM)], sem_i),
           pltpu.async_copy(nmB.at[pl.ds(astart_nm, F_NM)],
                            winNB.at[pl.ds(0, F_NM)], sem_i)]
    for cp in cps:
        cp.wait()

    lanes = lax.iota(jnp.int32, L)

    def transform(n_chunks, base, off, boundary, wa, wb, da, db):
        @plsc.parallel_loop(0, n_chunks, 1)
        def _(k):
            j0 = k * L
            p = base + j0 + lanes
            bias = jnp.where(p >= boundary, jnp.int32(N_PIX), jnp.int32(0))
            ra = wa[pl.ds(off + j0, L)]
            rb = wb[pl.ds(off + j0, L)]
            da[pl.ds(j0, L)] = jnp.clip(ra + bias, 0, ROW_MAX)
            db[pl.ds(j0, L)] = jnp.clip(rb + bias, 0, ROW_MAX)

    m_valid = jnp.minimum(M_PER_W, M_TOT - base_m)

    bufs = [(bufA0, bufB0, sem0), (bufA1, bufB1, sem1)]
    tiles = [(True, s, z) for (s, z) in M_TILES] + \
            [(False, s, z) for (s, z) in NM_TILES]

    def issue(i):
        is_m, start, size = tiles[i]
        bA, bB, sem = bufs[i % 2]
        iA = iAm_v if is_m else iAnm_v
        iB = iBm_v if is_m else iBnm_v
        h = (size // 2 + 7) // 8 * 8
        cps = []
        for (idx, buf) in ((iA, bA), (iB, bB)):
            cps.append(pltpu.async_copy(
                tableA.at[idx.at[pl.ds(start, h)]] if buf is bA else
                tableB.at[idx.at[pl.ds(start, h)]],
                buf.at[pl.ds(0, h)], sem))
            cps.append(pltpu.async_copy(
                tableA.at[idx.at[pl.ds(start + h, size - h)]] if buf is bA else
                tableB.at[idx.at[pl.ds(start + h, size - h)]],
                buf.at[pl.ds(h, size - h)], sem))
        return cps

    transform(F_M // L, base_m, off_m, N_MATCH, winMA, winMB, iAm_v, iBm_v)
    inflight = issue(0)
    transform(F_NM // L, base_nm, off_nm, N_NONMATCH, winNA, winNB,
              iAnm_v, iBnm_v)

    zero = jnp.zeros((L,), jnp.float32)

    def run_tile(bA, bB, n, accs, is_m):
        def body(r, acc):
            acc = list(acc)
            for j in range(D // L):
                a = bA[r, pl.ds(j * L, L)]
                b = bB[r, pl.ds(j * L, L)]
                d = a - b
                if is_m:
                    acc[j % 4] = acc[j % 4] + d * d
                else:
                    acc[j % 4] = acc[j % 4] + jnp.maximum(MARGIN - d * d, zero)
            return tuple(acc)
        return plsc.parallel_loop(0, n, 1, unroll=2, carry=accs)(body)

    acc_m = (zero, zero, zero, zero)
    acc_nm = (zero, zero, zero, zero)
    for i, (is_m, start, size) in enumerate(tiles):
        cur = inflight
        if i + 1 < len(tiles):
            inflight = issue(i + 1)
        for cp in cur:
            cp.wait()
        bA, bB, _ = bufs[i % 2]
        if is_m:
            n = jnp.clip(m_valid - start, 0, size)
            acc_m = run_tile(bA, bB, n, acc_m, True)
        else:
            acc_nm = run_tile(bA, bB, size, acc_nm, False)

    stage[pl.ds(0, L)] = (acc_m[0] + acc_m[1]) + (acc_m[2] + acc_m[3])
    stage[pl.ds(L, L)] = (acc_nm[0] + acc_nm[1]) + (acc_nm[2] + acc_nm[3])
    pltpu.sync_copy(stage, out.at[wid])


def kernel(outA, outB, matchA, matchB, nonMatchA, nonMatchB, hardNegative):
    i32 = jnp.int32
    parts = _sc_loss(
        outA.reshape(B * N_PIX, D),
        outB.reshape(B * N_PIX, D),
        matchA.astype(i32).reshape(M_TOT),
        matchB.astype(i32).reshape(M_TOT),
        nonMatchA.astype(i32).reshape(NM_TOT),
        nonMatchB.astype(i32).reshape(NM_TOT),
    )
    matchLossSum = parts[:, :L].sum() / N_MATCH
    nonMatchLossSum = NONMATCH_W * parts[:, L:].sum() / N_NONMATCH
    contrastiveLossSum = matchLossSum + nonMatchLossSum
    return (contrastiveLossSum, matchLossSum, nonMatchLossSum)


# speedup vs baseline: 1.0122x; 1.0122x over previous
import functools

import jax
import jax.numpy as jnp
from jax import lax
from jax.experimental import pallas as pl
from jax.experimental.pallas import tpu as pltpu
from jax.experimental.pallas import tpu_sc as plsc

B = 2
N_PIX = 147456
D = 128
N_MATCH = 5000
N_NONMATCH = 20000
MARGIN = 0.5
NONMATCH_W = 1.0

NC = 2
NS = 16
NW = NC * NS
L = 16

M_TOT = B * N_MATCH
NM_TOT = B * N_NONMATCH
M_PER_W = -(-M_TOT // NW)
NM_PER_W = NM_TOT // NW

F_M = 320
F_NM = 1264
ROW_MAX = B * N_PIX - 1

T = 128


def _tiles(total):
    out, s = [], 0
    while s < total:
        out.append((s, min(T, total - s)))
        s += T
    return out


M_TILES = _tiles(M_PER_W)
NM_TILES = _tiles(NM_PER_W)

_mesh = plsc.VectorSubcoreMesh(core_axis_name="c", subcore_axis_name="s")


@functools.partial(
    pl.kernel,
    out_type=jax.ShapeDtypeStruct((NW, 2 * L), jnp.float32),
    mesh=_mesh,
    scratch_types=[
        pltpu.VMEM((F_M + 2 * L,), jnp.int32),
        pltpu.VMEM((F_M + 2 * L,), jnp.int32),
        pltpu.VMEM((F_NM + L,), jnp.int32),
        pltpu.VMEM((F_NM + L,), jnp.int32),
        pltpu.VMEM((F_M,), jnp.int32),
        pltpu.VMEM((F_M,), jnp.int32),
        pltpu.VMEM((F_NM,), jnp.int32),
        pltpu.VMEM((F_NM,), jnp.int32),
        pltpu.VMEM((T, D), jnp.float32),
        pltpu.VMEM((T, D), jnp.float32),
        pltpu.VMEM((T, D), jnp.float32),
        pltpu.VMEM((T, D), jnp.float32),
        pltpu.VMEM((2 * L,), jnp.float32),
        pltpu.SemaphoreType.DMA,
        pltpu.SemaphoreType.DMA,
        pltpu.SemaphoreType.DMA,
    ],
)
def _sc_loss(tableA, tableB, mA, mB, nmA, nmB, out,
             winMA, winMB, winNA, winNB,
             iAm_v, iBm_v, iAnm_v, iBnm_v,
             bufA0, bufB0, bufA1, bufB1, stage, sem0, sem1, sem_i):
    wid = lax.axis_index("s") * NC + lax.axis_index("c")

    base_m = wid * M_PER_W
    astart_m = jnp.minimum((base_m // 8) * 8, M_TOT - F_M)
    off_m = base_m - astart_m
    base_nm = wid * NM_PER_W
    astart_nm = jnp.minimum((base_nm // 8) * 8, NM_TOT - F_NM)
    off_nm = base_nm - astart_nm

    cps = [pltpu.async_copy(mA.at[pl.ds(astart_m, F_M)],
                            winMA.at[pl.ds(0, F_M)], sem_i),
           pltpu.async_copy(mB.at[pl.ds(astart_m, F_M)],
                            winMB.at[pl.ds(0, F_M)], sem_i),
           pltpu.async_copy(nmA.at[pl.ds(astart_nm, F_NM)],
                            winNA.at[pl.ds(0, F_NM)], sem_i),
           pltpu.async_copy(nmB.at[pl.ds(astart_nm, F_NM)],
                            winNB.at[pl.ds(0, F_NM)], sem_i)]
    for cp in cps:
        cp.wait()

    lanes = lax.iota(jnp.int32, L)

    def transform(n_chunks, base, off, boundary, wa, wb, da, db):
        @plsc.parallel_loop(0, n_chunks, 1)
        def _(k):
            j0 = k * L
            p = base + j0 + lanes
            bias = jnp.where(p >= boundary, jnp.int32(N_PIX), jnp.int32(0))
            ra = wa[pl.ds(off + j0, L)]
            rb = wb[pl.ds(off + j0, L)]
            da[pl.ds(j0, L)] = jnp.clip(ra + bias, 0, ROW_MAX)
            db[pl.ds(j0, L)] = jnp.clip(rb + bias, 0, ROW_MAX)

    m_valid = jnp.minimum(M_PER_W, M_TOT - base_m)

    bufs = [(bufA0, bufB0, sem0), (bufA1, bufB1, sem1)]
    tiles = [(False, s, z) for (s, z) in NM_TILES] + \
            [(True, s, z) for (s, z) in M_TILES]

    def issue(i):
        is_m, start, size = tiles[i]
        bA, bB, sem = bufs[i % 2]
        ia = (iAm_v if is_m else iAnm_v).at[pl.ds(start, size)]
        ib = (iBm_v if is_m else iBnm_v).at[pl.ds(start, size)]
        cpA = pltpu.async_copy(tableA.at[ia], bA.at[pl.ds(0, size)], sem)
        cpB = pltpu.async_copy(tableB.at[ib], bB.at[pl.ds(0, size)], sem)
        return cpA, cpB

    transform(F_NM // L, base_nm, off_nm, N_NONMATCH, winNA, winNB,
              iAnm_v, iBnm_v)
    inflight = issue(0)
    transform(F_M // L, base_m, off_m, N_MATCH, winMA, winMB, iAm_v, iBm_v)

    zero = jnp.zeros((L,), jnp.float32)

    def run_tile(bA, bB, n, accs, is_m):
        def body(r, acc):
            acc = list(acc)
            for j in range(D // L):
                a = bA[r, pl.ds(j * L, L)]
                b = bB[r, pl.ds(j * L, L)]
                d = a - b
                if is_m:
                    acc[j % 4] = acc[j % 4] + d * d
                else:
                    acc[j % 4] = acc[j % 4] + jnp.maximum(MARGIN - d * d, zero)
            return tuple(acc)
        return plsc.parallel_loop(0, n, 1, unroll=2, carry=accs)(body)

    acc_m = (zero, zero, zero, zero)
    acc_nm = (zero, zero, zero, zero)
    for i, (is_m, start, size) in enumerate(tiles):
        cur = inflight
        if i + 1 < len(tiles):
            inflight = issue(i + 1)
        for cp in cur:
            cp.wait()
        bA, bB, _ = bufs[i % 2]
        if is_m:
            n = jnp.clip(m_valid - start, 0, size)
            acc_m = run_tile(bA, bB, n, acc_m, True)
        else:
            acc_nm = run_tile(bA, bB, size, acc_nm, False)

    stage[pl.ds(0, L)] = (acc_m[0] + acc_m[1]) + (acc_m[2] + acc_m[3])
    stage[pl.ds(L, L)] = (acc_nm[0] + acc_nm[1]) + (acc_nm[2] + acc_nm[3])
    pltpu.sync_copy(stage, out.at[wid])


def kernel(outA, outB, matchA, matchB, nonMatchA, nonMatchB, hardNegative):
    i32 = jnp.int32
    parts = _sc_loss(
        outA.reshape(B * N_PIX, D),
        outB.reshape(B * N_PIX, D),
        matchA.astype(i32).reshape(M_TOT),
        matchB.astype(i32).reshape(M_TOT),
        nonMatchA.astype(i32).reshape(NM_TOT),
        nonMatchB.astype(i32).reshape(NM_TOT),
    )
    matchLossSum = parts[:, :L].sum() / N_MATCH
    nonMatchLossSum = NONMATCH_W * parts[:, L:].sum() / N_NONMATCH
    contrastiveLossSum = matchLossSum + nonMatchLossSum
    return (contrastiveLossSum, matchLossSum, nonMatchLossSum)
